# baseline (speedup 1.0000x reference)
import jax
import jax.numpy as jnp
from jax.experimental import pallas as pl
from jax.experimental.pallas import tpu as pltpu

DEP = 128
SLOT = 128
GROUPS = 2
GROUP_SIZE = 1024
DB = 16


def _body(addr_ref, embt_ref, freq_ref, mem_ref, out_ref):
    a = addr_ref[...]
    ft = embt_ref[...] * freq_ref[...]
    contrib = jnp.sum(a * ft[:, :, None], axis=1)
    out_ref[...] = mem_ref[...] + contrib[None]


def kernel(batch_address, batch_embedding, batch_frequency, memory_matrix):
    embt = batch_embedding.T
    freq = batch_frequency[None, :]
    n_db = DEP // DB
    grid = (GROUPS, n_db)
    return pl.pallas_call(
        _body,
        grid=grid,
        in_specs=[
            pl.BlockSpec((DB, GROUP_SIZE, SLOT), lambda g, db: (db, g, 0)),
            pl.BlockSpec((DB, GROUP_SIZE), lambda g, db: (db, g)),
            pl.BlockSpec((1, GROUP_SIZE), lambda g, db: (0, g)),
            pl.BlockSpec((1, DB, SLOT), lambda g, db: (g, db, 0)),
        ],
        out_specs=pl.BlockSpec((1, DB, SLOT), lambda g, db: (g, db, 0)),
        out_shape=jax.ShapeDtypeStruct((GROUPS, DEP, SLOT), jnp.float32),
        compiler_params=pltpu.CompilerParams(
            dimension_semantics=("arbitrary", "arbitrary"),
        ),
    )(batch_address, embt, freq, memory_matrix)
